# baseline (speedup 1.0000x reference)
import functools
import math

import jax
import jax.numpy as jnp
from jax import lax
from jax.experimental import pallas as pl
from jax.experimental.pallas import tpu as pltpu
from jax.experimental.pallas import tpu_sc as plsc

NUM_EXPERTS = 64
TOKEN_BLOCK = 256
KEY_BIAS = 0x3C000000
AUX_W = 0.01
Z_W = 0.001


def _stage1_body(x_ref, w_ref, e_ref, k_ref, psum_ref, zsum_ref):
    i = pl.program_id(0)
    logits = lax.dot_general(
        x_ref[...], w_ref[...], (((1,), (1,)), ((), ())),
        preferred_element_type=jnp.float32)
    m = jnp.max(logits, axis=1, keepdims=True)
    ex = jnp.exp(logits - m)
    ssum = jnp.sum(ex, axis=1, keepdims=True)
    p = ex / ssum
    ei = jnp.argmax(p, axis=1).astype(jnp.int32)
    pm = jnp.max(p, axis=1)
    bits = lax.bitcast_convert_type(pm, jnp.int32)
    key = jnp.maximum(bits - KEY_BIAS, 1)
    e_ref[0, :, 0] = ei
    k_ref[0, :, 0] = key

    @pl.when(i == 0)
    def _():
        psum_ref[...] = jnp.zeros_like(psum_ref)
        zsum_ref[...] = jnp.zeros_like(zsum_ref)

    psum_ref[...] += jnp.sum(p, axis=0)[None, :]
    lse = m[:, 0] + jnp.log(ssum[:, 0])
    zsum_ref[...] += jnp.sum(lse * lse).reshape(1, 1)


def _sc_count_ge(list_ref, nchunks, u):
    def chunk(j, cnt):
        v = list_ref[pl.ds(j * 16, 16)]
        return cnt + plsc.all_reduce_population_count(v >= u)[0]
    return lax.fori_loop(0, nchunks, chunk, jnp.int32(0))


def _sc_search(list_ref, n, cap):
    nch = (n + 15) // 16
    def bit(i, t):
        cand = t | (jnp.int32(1) << (25 - i))
        c = _sc_count_ge(list_ref, nch, cand)
        return jnp.where(c >= cap, cand, t)
    t = lax.fori_loop(0, 26, bit, jnp.int32(0))
    c_ge = _sc_count_ge(list_ref, nch, t)
    c_gt = _sc_count_ge(list_ref, nch, t + 1)
    rem = jnp.where(t > 0, cap - c_gt, 0)
    kept = jnp.minimum(n, cap)
    simple = jnp.where(t > 0, (c_ge == cap).astype(jnp.int32), 1)
    tprime = jnp.maximum(t, 1)
    return tprime, t, rem, kept, simple


def _stage2_sc_body(capacity, ntok, e_hbm, k_hbm, out_hbm,
                    e_v, k_v, l0_v, l1_v, outbuf_v):
    cap = jnp.int32(capacity)
    wid = lax.axis_index("c") * 16 + lax.axis_index("s")
    ex0 = (wid * 2).astype(jnp.int32)
    ex1 = ex0 + 1
    pltpu.sync_copy(e_hbm, e_v)
    pltpu.sync_copy(k_hbm, k_v)

    def compact(j, carry):
        c0, c1 = carry
        ve = e_v[pl.ds(j * 16, 16)]
        vk = k_v[pl.ds(j * 16, 16)]
        m0 = ve == ex0
        m1 = ve == ex1
        plsc.store_compressed(l0_v.at[pl.ds(c0, 16)], vk, mask=m0)
        plsc.store_compressed(l1_v.at[pl.ds(c1, 16)], vk, mask=m1)
        c0 = c0 + plsc.all_reduce_population_count(m0)[0]
        c1 = c1 + plsc.all_reduce_population_count(m1)[0]
        return c0, c1

    n0, n1 = lax.fori_loop(0, ntok // 16, compact,
                           (jnp.int32(0), jnp.int32(0)))
    zeros = jnp.zeros((16,), jnp.int32)
    l0_v[pl.ds(n0, 16)] = zeros
    l1_v[pl.ds(n1, 16)] = zeros
    tp0, t0, rem0, kept0, sim0 = _sc_search(l0_v, n0, cap)
    tp1, t1, rem1, kept1, sim1 = _sc_search(l1_v, n1, cap)

    lane = lax.iota(jnp.int32, 16)
    outv = jnp.where(lane == 0, tp0,
           jnp.where(lane == 1, tp1,
           jnp.where(lane == 2, t0,
           jnp.where(lane == 3, t1,
           jnp.where(lane == 4, rem0,
           jnp.where(lane == 5, rem1,
           jnp.where(lane == 6, kept0,
           jnp.where(lane == 7, kept1,
           jnp.where(lane == 8, sim0,
           jnp.where(lane == 9, sim1, 0))))))))))
    outbuf_v[...] = outv
    pltpu.sync_copy(outbuf_v, out_hbm.at[wid])


def _stage3_body(total_tokens, e_ref, k_ref, tp_ref, t_ref, rem_ref,
                 kept_ref, sim_ref, psum_ref, zsum_ref,
                 mask_ref, loss_ref, carry_ref):
    i = pl.program_id(0)

    @pl.when(i == 0)
    def _():
        carry_ref[...] = jnp.zeros_like(carry_ref)

    ec = e_ref[0, :, :]
    kc = k_ref[0, :, :]
    iota_e = lax.broadcasted_iota(jnp.int32, (1, NUM_EXPERTS), 1)
    assigned = ec == iota_e
    S = jnp.where(assigned, kc, 0)
    all_simple = jnp.min(sim_ref[...]) == 1

    @pl.when(all_simple)
    def _():
        mask_ref[...] = (S >= tp_ref[...]).astype(jnp.float32)

    @pl.when(jnp.logical_not(all_simple))
    def _():
        t = t_ref[...]
        gt = S > t
        eq = (S == t) & (t > 0)
        eqf = eq.astype(jnp.float32)
        row = lax.broadcasted_iota(jnp.int32, (TOKEN_BLOCK, TOKEN_BLOCK), 0)
        col = lax.broadcasted_iota(jnp.int32, (TOKEN_BLOCK, TOKEN_BLOCK), 1)
        tril = (row > col).astype(jnp.float32)
        excl = lax.dot_general(tril, eqf, (((1,), (0,)), ((), ())),
                               preferred_element_type=jnp.float32)
        rank = carry_ref[...] + excl
        keep_eq = eq & (rank < rem_ref[...].astype(jnp.float32))
        mask_ref[...] = (gt | keep_eq).astype(jnp.float32)
        carry_ref[...] += jnp.sum(eqf, axis=0)[None, :]

    @pl.when(i == pl.num_programs(0) - 1)
    def _():
        n = jnp.float32(total_tokens)
        f = kept_ref[...].astype(jnp.float32) / n
        pmean = psum_ref[...] / n
        aux = AUX_W * jnp.sum(f * pmean) * NUM_EXPERTS
        z = Z_W * zsum_ref[...] / n
        loss_ref[...] = aux + z


def kernel(x, W):
    Bb, Tt, C = x.shape
    E = W.shape[0]
    n = Bb * Tt
    capacity = math.ceil(n / E)
    xr = x.reshape(n, C)
    nblk = n // TOKEN_BLOCK

    e_col, k_col, psum, zsum = pl.pallas_call(
        _stage1_body,
        grid=(nblk,),
        in_specs=[
            pl.BlockSpec((TOKEN_BLOCK, C), lambda i: (i, 0)),
            pl.BlockSpec((E, C), lambda i: (0, 0)),
        ],
        out_specs=[
            pl.BlockSpec((1, TOKEN_BLOCK, 1), lambda i: (i, 0, 0)),
            pl.BlockSpec((1, TOKEN_BLOCK, 1), lambda i: (i, 0, 0)),
            pl.BlockSpec((1, E), lambda i: (0, 0)),
            pl.BlockSpec((1, 1), lambda i: (0, 0)),
        ],
        out_shape=[
            jax.ShapeDtypeStruct((nblk, TOKEN_BLOCK, 1), jnp.int32),
            jax.ShapeDtypeStruct((nblk, TOKEN_BLOCK, 1), jnp.int32),
            jax.ShapeDtypeStruct((1, E), jnp.float32),
            jax.ShapeDtypeStruct((1, 1), jnp.float32),
        ],
    )(xr, W)

    sc_out = pl.kernel(
        functools.partial(_stage2_sc_body, capacity, n),
        out_type=jax.ShapeDtypeStruct((32, 16), jnp.int32),
        mesh=plsc.VectorSubcoreMesh(core_axis_name="c", subcore_axis_name="s"),
        compiler_params=pltpu.CompilerParams(needs_layout_passes=False),
        scratch_types=[
            pltpu.VMEM((n,), jnp.int32),
            pltpu.VMEM((n,), jnp.int32),
            pltpu.VMEM((n + 16,), jnp.int32),
            pltpu.VMEM((n + 16,), jnp.int32),
            pltpu.VMEM((16,), jnp.int32),
        ],
    )(e_col.reshape(n), k_col.reshape(n))
    tprime = sc_out[:, 0:2].reshape(1, E)
    t = sc_out[:, 2:4].reshape(1, E)
    rem = sc_out[:, 4:6].reshape(1, E)
    kept = sc_out[:, 6:8].reshape(1, E)
    simple = sc_out[:, 8:10].reshape(1, E)

    mask, loss = pl.pallas_call(
        functools.partial(_stage3_body, n),
        grid=(nblk,),
        in_specs=[
            pl.BlockSpec((1, TOKEN_BLOCK, 1), lambda i: (i, 0, 0)),
            pl.BlockSpec((1, TOKEN_BLOCK, 1), lambda i: (i, 0, 0)),
            pl.BlockSpec((1, E), lambda i: (0, 0)),
            pl.BlockSpec((1, E), lambda i: (0, 0)),
            pl.BlockSpec((1, E), lambda i: (0, 0)),
            pl.BlockSpec((1, E), lambda i: (0, 0)),
            pl.BlockSpec((1, E), lambda i: (0, 0)),
            pl.BlockSpec((1, E), lambda i: (0, 0)),
            pl.BlockSpec((1, 1), lambda i: (0, 0)),
        ],
        out_specs=[
            pl.BlockSpec((TOKEN_BLOCK, E), lambda i: (i, 0)),
            pl.BlockSpec((1, 1), lambda i: (0, 0)),
        ],
        out_shape=[
            jax.ShapeDtypeStruct((n, E), jnp.float32),
            jax.ShapeDtypeStruct((1, 1), jnp.float32),
        ],
        scratch_shapes=[pltpu.VMEM((1, E), jnp.float32)],
    )(e_col, k_col, tprime, t, rem, kept, simple, psum, zsum)

    mask = mask.reshape(Bb, Tt, E)
    return mask, mask, loss.reshape(())


# speedup vs baseline: 1.0375x; 1.0375x over previous
import functools
import math

import jax
import jax.numpy as jnp
from jax import lax
from jax.experimental import pallas as pl
from jax.experimental.pallas import tpu as pltpu
from jax.experimental.pallas import tpu_sc as plsc

NUM_EXPERTS = 64
TOKEN_BLOCK = 256
KEY_BIAS = 0x3C000000
AUX_W = 0.01
Z_W = 0.001


def _stage1_body(x_ref, w_ref, s_ref, e_ref, k_ref, psum_ref, zsum_ref):
    i = pl.program_id(0)
    logits = lax.dot_general(
        x_ref[...], w_ref[...], (((1,), (1,)), ((), ())),
        preferred_element_type=jnp.float32)
    m = jnp.max(logits, axis=1, keepdims=True)
    ex = jnp.exp(logits - m)
    ssum = jnp.sum(ex, axis=1, keepdims=True)
    p = ex / ssum
    ei = jnp.argmax(p, axis=1).astype(jnp.int32)
    pm = jnp.max(p, axis=1)
    bits = lax.bitcast_convert_type(pm, jnp.int32)
    key = jnp.maximum(bits - KEY_BIAS, 1)
    onehot = ei[:, None] == lax.broadcasted_iota(jnp.int32, (1, NUM_EXPERTS), 1)
    s_ref[...] = jnp.where(onehot, key[:, None], 0)
    e_ref[0, 0, :] = ei
    k_ref[0, 0, :] = key

    @pl.when(i == 0)
    def _():
        psum_ref[...] = jnp.zeros_like(psum_ref)
        zsum_ref[...] = jnp.zeros_like(zsum_ref)

    psum_ref[...] += jnp.sum(p, axis=0)[None, :]
    lse = m[:, 0] + jnp.log(ssum[:, 0])
    zsum_ref[...] += jnp.sum(lse * lse).reshape(1, 1)


def _sc_count_ge(list_ref, nchunks, u):
    def chunk(j, cnt):
        v = list_ref[pl.ds(j * 16, 16)]
        return cnt + plsc.all_reduce_population_count(v >= u)[0]
    return lax.fori_loop(0, nchunks, chunk, jnp.int32(0))


def _sc_search(list_ref, n, cap):
    nch = (n + 15) // 16
    def bit(i, t):
        cand = t | (jnp.int32(1) << (25 - i))
        c = _sc_count_ge(list_ref, nch, cand)
        return jnp.where(c >= cap, cand, t)
    t = lax.fori_loop(0, 26, bit, jnp.int32(0))
    c_ge = _sc_count_ge(list_ref, nch, t)
    c_gt = _sc_count_ge(list_ref, nch, t + 1)
    rem = jnp.where(t > 0, cap - c_gt, 0)
    kept = jnp.minimum(n, cap)
    simple = jnp.where(t > 0, (c_ge == cap).astype(jnp.int32), 1)
    tprime = jnp.maximum(t, 1)
    return tprime, t, rem, kept, simple


def _stage2_sc_body(capacity, ntok, e_hbm, k_hbm, out_hbm,
                    e_v, k_v, l0_v, l1_v, outbuf_v):
    cap = jnp.int32(capacity)
    wid = lax.axis_index("c") * 16 + lax.axis_index("s")
    ex0 = (wid * 2).astype(jnp.int32)
    ex1 = ex0 + 1
    pltpu.sync_copy(e_hbm, e_v)
    pltpu.sync_copy(k_hbm, k_v)

    def compact(j, carry):
        c0, c1 = carry
        ve = e_v[pl.ds(j * 16, 16)]
        vk = k_v[pl.ds(j * 16, 16)]
        m0 = ve == ex0
        m1 = ve == ex1
        plsc.store_compressed(l0_v.at[pl.ds(c0, 16)], vk, mask=m0)
        plsc.store_compressed(l1_v.at[pl.ds(c1, 16)], vk, mask=m1)
        c0 = c0 + plsc.all_reduce_population_count(m0)[0]
        c1 = c1 + plsc.all_reduce_population_count(m1)[0]
        return c0, c1

    n0, n1 = lax.fori_loop(0, ntok // 16, compact,
                           (jnp.int32(0), jnp.int32(0)))
    zeros = jnp.zeros((16,), jnp.int32)
    l0_v[pl.ds(n0, 16)] = zeros
    l1_v[pl.ds(n1, 16)] = zeros
    tp0, t0, rem0, kept0, sim0 = _sc_search(l0_v, n0, cap)
    tp1, t1, rem1, kept1, sim1 = _sc_search(l1_v, n1, cap)

    lane = lax.iota(jnp.int32, 16)
    outv = jnp.where(lane == 0, tp0,
           jnp.where(lane == 1, tp1,
           jnp.where(lane == 2, t0,
           jnp.where(lane == 3, t1,
           jnp.where(lane == 4, rem0,
           jnp.where(lane == 5, rem1,
           jnp.where(lane == 6, kept0,
           jnp.where(lane == 7, kept1,
           jnp.where(lane == 8, sim0,
           jnp.where(lane == 9, sim1, 0))))))))))
    outbuf_v[...] = outv
    pltpu.sync_copy(outbuf_v, out_hbm.at[wid])


def _stage3_body(total_tokens, s_ref, tp_ref, t_ref, rem_ref,
                 kept_ref, sim_ref, psum_ref, zsum_ref,
                 mask_ref, loss_ref, carry_ref):
    i = pl.program_id(0)

    @pl.when(i == 0)
    def _():
        carry_ref[...] = jnp.zeros_like(carry_ref)

    S = s_ref[...]
    all_simple = jnp.min(sim_ref[...]) == 1

    @pl.when(all_simple)
    def _():
        mask_ref[...] = (S >= tp_ref[...]).astype(jnp.float32)

    @pl.when(jnp.logical_not(all_simple))
    def _():
        t = t_ref[...]
        gt = S > t
        eq = (S == t) & (t > 0)
        eqf = eq.astype(jnp.float32)
        row = lax.broadcasted_iota(jnp.int32, (TOKEN_BLOCK, TOKEN_BLOCK), 0)
        col = lax.broadcasted_iota(jnp.int32, (TOKEN_BLOCK, TOKEN_BLOCK), 1)
        tril = (row > col).astype(jnp.float32)
        excl = lax.dot_general(tril, eqf, (((1,), (0,)), ((), ())),
                               preferred_element_type=jnp.float32)
        rank = carry_ref[...] + excl
        keep_eq = eq & (rank < rem_ref[...].astype(jnp.float32))
        mask_ref[...] = (gt | keep_eq).astype(jnp.float32)
        carry_ref[...] += jnp.sum(eqf, axis=0)[None, :]

    @pl.when(i == pl.num_programs(0) - 1)
    def _():
        n = jnp.float32(total_tokens)
        f = kept_ref[...].astype(jnp.float32) / n
        pmean = psum_ref[...] / n
        aux = AUX_W * jnp.sum(f * pmean) * NUM_EXPERTS
        z = Z_W * zsum_ref[...] / n
        loss_ref[...] = aux + z


def kernel(x, W):
    Bb, Tt, C = x.shape
    E = W.shape[0]
    n = Bb * Tt
    capacity = math.ceil(n / E)
    xr = x.reshape(n, C)
    nblk = n // TOKEN_BLOCK

    S, e_row, k_row, psum, zsum = pl.pallas_call(
        _stage1_body,
        grid=(nblk,),
        in_specs=[
            pl.BlockSpec((TOKEN_BLOCK, C), lambda i: (i, 0)),
            pl.BlockSpec((E, C), lambda i: (0, 0)),
        ],
        out_specs=[
            pl.BlockSpec((TOKEN_BLOCK, E), lambda i: (i, 0)),
            pl.BlockSpec((1, 1, TOKEN_BLOCK), lambda i: (i, 0, 0)),
            pl.BlockSpec((1, 1, TOKEN_BLOCK), lambda i: (i, 0, 0)),
            pl.BlockSpec((1, E), lambda i: (0, 0)),
            pl.BlockSpec((1, 1), lambda i: (0, 0)),
        ],
        out_shape=[
            jax.ShapeDtypeStruct((n, E), jnp.int32),
            jax.ShapeDtypeStruct((nblk, 1, TOKEN_BLOCK), jnp.int32),
            jax.ShapeDtypeStruct((nblk, 1, TOKEN_BLOCK), jnp.int32),
            jax.ShapeDtypeStruct((1, E), jnp.float32),
            jax.ShapeDtypeStruct((1, 1), jnp.float32),
        ],
    )(xr, W)

    sc_out = pl.kernel(
        functools.partial(_stage2_sc_body, capacity, n),
        out_type=jax.ShapeDtypeStruct((32, 16), jnp.int32),
        mesh=plsc.VectorSubcoreMesh(core_axis_name="c", subcore_axis_name="s"),
        compiler_params=pltpu.CompilerParams(needs_layout_passes=False),
        scratch_types=[
            pltpu.VMEM((n,), jnp.int32),
            pltpu.VMEM((n,), jnp.int32),
            pltpu.VMEM((n + 16,), jnp.int32),
            pltpu.VMEM((n + 16,), jnp.int32),
            pltpu.VMEM((16,), jnp.int32),
        ],
    )(e_row.reshape(n), k_row.reshape(n))
    tprime = sc_out[:, 0:2].reshape(1, E)
    t = sc_out[:, 2:4].reshape(1, E)
    rem = sc_out[:, 4:6].reshape(1, E)
    kept = sc_out[:, 6:8].reshape(1, E)
    simple = sc_out[:, 8:10].reshape(1, E)

    mask, loss = pl.pallas_call(
        functools.partial(_stage3_body, n),
        grid=(nblk,),
        in_specs=[
            pl.BlockSpec((TOKEN_BLOCK, E), lambda i: (i, 0)),
            pl.BlockSpec((1, E), lambda i: (0, 0)),
            pl.BlockSpec((1, E), lambda i: (0, 0)),
            pl.BlockSpec((1, E), lambda i: (0, 0)),
            pl.BlockSpec((1, E), lambda i: (0, 0)),
            pl.BlockSpec((1, E), lambda i: (0, 0)),
            pl.BlockSpec((1, E), lambda i: (0, 0)),
            pl.BlockSpec((1, 1), lambda i: (0, 0)),
        ],
        out_specs=[
            pl.BlockSpec((TOKEN_BLOCK, E), lambda i: (i, 0)),
            pl.BlockSpec((1, 1), lambda i: (0, 0)),
        ],
        out_shape=[
            jax.ShapeDtypeStruct((n, E), jnp.float32),
            jax.ShapeDtypeStruct((1, 1), jnp.float32),
        ],
        scratch_shapes=[pltpu.VMEM((1, E), jnp.float32)],
    )(S, tprime, t, rem, kept, simple, psum, zsum)

    mask = mask.reshape(Bb, Tt, E)
    return mask, mask, loss.reshape(())


# speedup vs baseline: 1.2892x; 1.2425x over previous
import functools
import math

import jax
import jax.numpy as jnp
from jax import lax
from jax.experimental import pallas as pl
from jax.experimental.pallas import tpu as pltpu
from jax.experimental.pallas import tpu_sc as plsc

NUM_EXPERTS = 64
TOKEN_BLOCK = 512
KEY_BIAS = 0x3C000000
AUX_W = 0.01
Z_W = 0.001


def _stage1_body(x_ref, w_ref, s_ref, e_ref, k_ref, psum_ref, zsum_ref):
    i = pl.program_id(0)
    logits = lax.dot_general(
        x_ref[...], w_ref[...], (((1,), (1,)), ((), ())),
        preferred_element_type=jnp.float32)
    m = jnp.max(logits, axis=1, keepdims=True)
    ex = jnp.exp(logits - m)
    ssum = jnp.sum(ex, axis=1, keepdims=True)
    p = ex / ssum
    ei = jnp.argmax(p, axis=1).astype(jnp.int32)
    pm = jnp.max(p, axis=1)
    bits = lax.bitcast_convert_type(pm, jnp.int32)
    key = jnp.maximum(bits - KEY_BIAS, 1)
    onehot = ei[:, None] == lax.broadcasted_iota(jnp.int32, (1, NUM_EXPERTS), 1)
    s_ref[...] = jnp.where(onehot, key[:, None], 0)
    e_ref[0, 0, :] = ei
    k_ref[0, 0, :] = key

    @pl.when(i == 0)
    def _():
        psum_ref[...] = jnp.zeros_like(psum_ref)
        zsum_ref[...] = jnp.zeros_like(zsum_ref)

    psum_ref[...] += jnp.sum(p, axis=0)[None, :]
    lse = m[:, 0] + jnp.log(ssum[:, 0])
    zsum_ref[...] += jnp.sum(lse * lse).reshape(1, 1)


def _sc_count_ge(list_ref, nchunks, u):
    def chunk(j, cnt):
        v = list_ref[pl.ds(j * 16, 16)]
        return cnt + plsc.all_reduce_population_count(v >= u)[0]
    return lax.fori_loop(0, nchunks, chunk, jnp.int32(0))


def _sc_search(list_ref, n, cap):
    nch = (n + 15) // 16
    def bit(i, t):
        cand = t | (jnp.int32(1) << (25 - i))
        c = _sc_count_ge(list_ref, nch, cand)
        return jnp.where(c >= cap, cand, t)
    t = lax.fori_loop(0, 26, bit, jnp.int32(0))
    c_ge = _sc_count_ge(list_ref, nch, t)
    c_gt = _sc_count_ge(list_ref, nch, t + 1)
    rem = jnp.where(t > 0, cap - c_gt, 0)
    kept = jnp.minimum(n, cap)
    simple = jnp.where(t > 0, (c_ge == cap).astype(jnp.int32), 1)
    tprime = jnp.maximum(t, 1)
    return tprime, t, rem, kept, simple


def _stage2_sc_body(capacity, ntok, e_hbm, k_hbm, out_hbm,
                    e_v, k_v, l0_v, l1_v, outbuf_v):
    cap = jnp.int32(capacity)
    wid = lax.axis_index("c") * 16 + lax.axis_index("s")
    ex0 = (wid * 2).astype(jnp.int32)
    ex1 = ex0 + 1
    pltpu.sync_copy(e_hbm, e_v)
    pltpu.sync_copy(k_hbm, k_v)

    def compact(j, carry):
        c0, c1 = carry
        ve = e_v[pl.ds(j * 16, 16)]
        vk = k_v[pl.ds(j * 16, 16)]
        m0 = ve == ex0
        m1 = ve == ex1
        plsc.store_compressed(l0_v.at[pl.ds(c0, 16)], vk, mask=m0)
        plsc.store_compressed(l1_v.at[pl.ds(c1, 16)], vk, mask=m1)
        c0 = c0 + plsc.all_reduce_population_count(m0)[0]
        c1 = c1 + plsc.all_reduce_population_count(m1)[0]
        return c0, c1

    n0, n1 = lax.fori_loop(0, ntok // 16, compact,
                           (jnp.int32(0), jnp.int32(0)))
    zeros = jnp.zeros((16,), jnp.int32)
    l0_v[pl.ds(n0, 16)] = zeros
    l1_v[pl.ds(n1, 16)] = zeros
    tp0, t0, rem0, kept0, sim0 = _sc_search(l0_v, n0, cap)
    tp1, t1, rem1, kept1, sim1 = _sc_search(l1_v, n1, cap)

    lane = lax.iota(jnp.int32, 16)
    outv = jnp.where(lane == 0, tp0,
           jnp.where(lane == 1, tp1,
           jnp.where(lane == 2, t0,
           jnp.where(lane == 3, t1,
           jnp.where(lane == 4, rem0,
           jnp.where(lane == 5, rem1,
           jnp.where(lane == 6, kept0,
           jnp.where(lane == 7, kept1,
           jnp.where(lane == 8, sim0,
           jnp.where(lane == 9, sim1, 0))))))))))
    outbuf_v[...] = outv
    pltpu.sync_copy(outbuf_v, out_hbm.at[wid])


def _stage3_body(total_tokens, s_ref, tp_ref, t_ref, rem_ref,
                 kept_ref, sim_ref, psum_ref, zsum_ref,
                 mask_ref, loss_ref, carry_ref):
    i = pl.program_id(0)

    @pl.when(i == 0)
    def _():
        carry_ref[...] = jnp.zeros_like(carry_ref)

    S = s_ref[...]
    all_simple = jnp.min(sim_ref[...]) == 1

    @pl.when(all_simple)
    def _():
        mask_ref[...] = (S >= tp_ref[...]).astype(jnp.float32)

    @pl.when(jnp.logical_not(all_simple))
    def _():
        t = t_ref[...]
        gt = S > t
        eq = (S == t) & (t > 0)
        eqf = eq.astype(jnp.float32)
        row = lax.broadcasted_iota(jnp.int32, (TOKEN_BLOCK, TOKEN_BLOCK), 0)
        col = lax.broadcasted_iota(jnp.int32, (TOKEN_BLOCK, TOKEN_BLOCK), 1)
        tril = (row > col).astype(jnp.float32)
        excl = lax.dot_general(tril, eqf, (((1,), (0,)), ((), ())),
                               preferred_element_type=jnp.float32)
        rank = carry_ref[...] + excl
        keep_eq = eq & (rank < rem_ref[...].astype(jnp.float32))
        mask_ref[...] = (gt | keep_eq).astype(jnp.float32)
        carry_ref[...] += jnp.sum(eqf, axis=0)[None, :]

    @pl.when(i == pl.num_programs(0) - 1)
    def _():
        n = jnp.float32(total_tokens)
        f = kept_ref[...].astype(jnp.float32) / n
        pmean = psum_ref[...] / n
        aux = AUX_W * jnp.sum(f * pmean) * NUM_EXPERTS
        z = Z_W * zsum_ref[...] / n
        loss_ref[...] = aux + z


def kernel(x, W):
    Bb, Tt, C = x.shape
    E = W.shape[0]
    n = Bb * Tt
    capacity = math.ceil(n / E)
    xr = x.reshape(n, C)
    nblk = n // TOKEN_BLOCK

    S, e_row, k_row, psum, zsum = pl.pallas_call(
        _stage1_body,
        grid=(nblk,),
        in_specs=[
            pl.BlockSpec((TOKEN_BLOCK, C), lambda i: (i, 0)),
            pl.BlockSpec((E, C), lambda i: (0, 0)),
        ],
        out_specs=[
            pl.BlockSpec((TOKEN_BLOCK, E), lambda i: (i, 0)),
            pl.BlockSpec((1, 1, TOKEN_BLOCK), lambda i: (i, 0, 0)),
            pl.BlockSpec((1, 1, TOKEN_BLOCK), lambda i: (i, 0, 0)),
            pl.BlockSpec((1, E), lambda i: (0, 0)),
            pl.BlockSpec((1, 1), lambda i: (0, 0)),
        ],
        out_shape=[
            jax.ShapeDtypeStruct((n, E), jnp.int32),
            jax.ShapeDtypeStruct((nblk, 1, TOKEN_BLOCK), jnp.int32),
            jax.ShapeDtypeStruct((nblk, 1, TOKEN_BLOCK), jnp.int32),
            jax.ShapeDtypeStruct((1, E), jnp.float32),
            jax.ShapeDtypeStruct((1, 1), jnp.float32),
        ],
    )(xr, W)

    sc_out = pl.kernel(
        functools.partial(_stage2_sc_body, capacity, n),
        out_type=jax.ShapeDtypeStruct((32, 16), jnp.int32),
        mesh=plsc.VectorSubcoreMesh(core_axis_name="c", subcore_axis_name="s"),
        compiler_params=pltpu.CompilerParams(needs_layout_passes=False),
        scratch_types=[
            pltpu.VMEM((n,), jnp.int32),
            pltpu.VMEM((n,), jnp.int32),
            pltpu.VMEM((n + 16,), jnp.int32),
            pltpu.VMEM((n + 16,), jnp.int32),
            pltpu.VMEM((16,), jnp.int32),
        ],
    )(e_row.reshape(n), k_row.reshape(n))
    tprime = sc_out[:, 0:2].reshape(1, E)
    t = sc_out[:, 2:4].reshape(1, E)
    rem = sc_out[:, 4:6].reshape(1, E)
    kept = sc_out[:, 6:8].reshape(1, E)
    simple = sc_out[:, 8:10].reshape(1, E)

    mask, loss = pl.pallas_call(
        functools.partial(_stage3_body, n),
        grid=(nblk,),
        in_specs=[
            pl.BlockSpec((TOKEN_BLOCK, E), lambda i: (i, 0)),
            pl.BlockSpec((1, E), lambda i: (0, 0)),
            pl.BlockSpec((1, E), lambda i: (0, 0)),
            pl.BlockSpec((1, E), lambda i: (0, 0)),
            pl.BlockSpec((1, E), lambda i: (0, 0)),
            pl.BlockSpec((1, E), lambda i: (0, 0)),
            pl.BlockSpec((1, E), lambda i: (0, 0)),
            pl.BlockSpec((1, 1), lambda i: (0, 0)),
        ],
        out_specs=[
            pl.BlockSpec((TOKEN_BLOCK, E), lambda i: (i, 0)),
            pl.BlockSpec((1, 1), lambda i: (0, 0)),
        ],
        out_shape=[
            jax.ShapeDtypeStruct((n, E), jnp.float32),
            jax.ShapeDtypeStruct((1, 1), jnp.float32),
        ],
        scratch_shapes=[pltpu.VMEM((1, E), jnp.float32)],
    )(S, tprime, t, rem, kept, simple, psum, zsum)

    mask = mask.reshape(Bb, Tt, E)
    return mask, mask, loss.reshape(())


# speedup vs baseline: 1.3567x; 1.0524x over previous
import functools
import math

import jax
import jax.numpy as jnp
from jax import lax
from jax.experimental import pallas as pl
from jax.experimental.pallas import tpu as pltpu
from jax.experimental.pallas import tpu_sc as plsc

NUM_EXPERTS = 64
TOKEN_BLOCK = 1024
KEY_BIAS = 0x3C000000
AUX_W = 0.01
Z_W = 0.001


def _stage1_body(x_ref, w_ref, s_ref, e_ref, k_ref, psum_ref, zsum_ref):
    i = pl.program_id(0)
    logits = lax.dot_general(
        x_ref[...], w_ref[...], (((1,), (1,)), ((), ())),
        preferred_element_type=jnp.float32)
    m = jnp.max(logits, axis=1, keepdims=True)
    ex = jnp.exp(logits - m)
    ssum = jnp.sum(ex, axis=1, keepdims=True)
    p = ex / ssum
    ei = jnp.argmax(p, axis=1).astype(jnp.int32)
    pm = jnp.max(p, axis=1)
    bits = lax.bitcast_convert_type(pm, jnp.int32)
    key = jnp.maximum(bits - KEY_BIAS, 1)
    onehot = ei[:, None] == lax.broadcasted_iota(jnp.int32, (1, NUM_EXPERTS), 1)
    s_ref[...] = jnp.where(onehot, key[:, None], 0)
    e_ref[0, 0, :] = ei
    k_ref[0, 0, :] = key

    @pl.when(i == 0)
    def _():
        psum_ref[...] = jnp.zeros_like(psum_ref)
        zsum_ref[...] = jnp.zeros_like(zsum_ref)

    psum_ref[...] += jnp.sum(p, axis=0)[None, :]
    lse = m[:, 0] + jnp.log(ssum[:, 0])
    zsum_ref[...] += jnp.sum(lse * lse).reshape(1, 1)


def _sc_count_ge(list_ref, nchunks, u):
    def chunk(j, cnt):
        v = list_ref[pl.ds(j * 16, 16)]
        return cnt + plsc.all_reduce_population_count(v >= u)[0]
    return lax.fori_loop(0, nchunks, chunk, jnp.int32(0))


def _sc_search(list_ref, n, cap):
    nch = (n + 15) // 16
    def bit(i, t):
        cand = t | (jnp.int32(1) << (25 - i))
        c = _sc_count_ge(list_ref, nch, cand)
        return jnp.where(c >= cap, cand, t)
    t = lax.fori_loop(0, 26, bit, jnp.int32(0))
    c_ge = _sc_count_ge(list_ref, nch, t)
    c_gt = _sc_count_ge(list_ref, nch, t + 1)
    rem = jnp.where(t > 0, cap - c_gt, 0)
    kept = jnp.minimum(n, cap)
    simple = jnp.where(t > 0, (c_ge == cap).astype(jnp.int32), 1)
    tprime = jnp.maximum(t, 1)
    return tprime, t, rem, kept, simple


def _stage2_sc_body(capacity, ntok, e_hbm, k_hbm, out_hbm,
                    e_v, k_v, l0_v, l1_v, outbuf_v):
    cap = jnp.int32(capacity)
    wid = lax.axis_index("c") * 16 + lax.axis_index("s")
    ex0 = (wid * 2).astype(jnp.int32)
    ex1 = ex0 + 1
    pltpu.sync_copy(e_hbm, e_v)
    pltpu.sync_copy(k_hbm, k_v)

    def compact(j, carry):
        c0, c1 = carry
        ve = e_v[pl.ds(j * 16, 16)]
        vk = k_v[pl.ds(j * 16, 16)]
        m0 = ve == ex0
        m1 = ve == ex1
        plsc.store_compressed(l0_v.at[pl.ds(c0, 16)], vk, mask=m0)
        plsc.store_compressed(l1_v.at[pl.ds(c1, 16)], vk, mask=m1)
        c0 = c0 + plsc.all_reduce_population_count(m0)[0]
        c1 = c1 + plsc.all_reduce_population_count(m1)[0]
        return c0, c1

    n0, n1 = lax.fori_loop(0, ntok // 16, compact,
                           (jnp.int32(0), jnp.int32(0)))
    zeros = jnp.zeros((16,), jnp.int32)
    l0_v[pl.ds(n0, 16)] = zeros
    l1_v[pl.ds(n1, 16)] = zeros
    tp0, t0, rem0, kept0, sim0 = _sc_search(l0_v, n0, cap)
    tp1, t1, rem1, kept1, sim1 = _sc_search(l1_v, n1, cap)

    lane = lax.iota(jnp.int32, 16)
    outv = jnp.where(lane == 0, tp0,
           jnp.where(lane == 1, tp1,
           jnp.where(lane == 2, t0,
           jnp.where(lane == 3, t1,
           jnp.where(lane == 4, rem0,
           jnp.where(lane == 5, rem1,
           jnp.where(lane == 6, kept0,
           jnp.where(lane == 7, kept1,
           jnp.where(lane == 8, sim0,
           jnp.where(lane == 9, sim1, 0))))))))))
    outbuf_v[...] = outv
    pltpu.sync_copy(outbuf_v, out_hbm.at[wid])


def _stage3_body(total_tokens, s_ref, tp_ref, t_ref, rem_ref,
                 kept_ref, sim_ref, psum_ref, zsum_ref,
                 mask_ref, loss_ref, carry_ref):
    i = pl.program_id(0)

    @pl.when(i == 0)
    def _():
        carry_ref[...] = jnp.zeros_like(carry_ref)

    S = s_ref[...]
    all_simple = jnp.min(sim_ref[...]) == 1

    @pl.when(all_simple)
    def _():
        mask_ref[...] = (S >= tp_ref[...]).astype(jnp.float32)

    @pl.when(jnp.logical_not(all_simple))
    def _():
        t = t_ref[...]
        gt = S > t
        eq = (S == t) & (t > 0)
        eqf = eq.astype(jnp.float32)
        row = lax.broadcasted_iota(jnp.int32, (TOKEN_BLOCK, TOKEN_BLOCK), 0)
        col = lax.broadcasted_iota(jnp.int32, (TOKEN_BLOCK, TOKEN_BLOCK), 1)
        tril = (row > col).astype(jnp.float32)
        excl = lax.dot_general(tril, eqf, (((1,), (0,)), ((), ())),
                               preferred_element_type=jnp.float32)
        rank = carry_ref[...] + excl
        keep_eq = eq & (rank < rem_ref[...].astype(jnp.float32))
        mask_ref[...] = (gt | keep_eq).astype(jnp.float32)
        carry_ref[...] += jnp.sum(eqf, axis=0)[None, :]

    @pl.when(i == pl.num_programs(0) - 1)
    def _():
        n = jnp.float32(total_tokens)
        f = kept_ref[...].astype(jnp.float32) / n
        pmean = psum_ref[...] / n
        aux = AUX_W * jnp.sum(f * pmean) * NUM_EXPERTS
        z = Z_W * zsum_ref[...] / n
        loss_ref[...] = aux + z


def kernel(x, W):
    Bb, Tt, C = x.shape
    E = W.shape[0]
    n = Bb * Tt
    capacity = math.ceil(n / E)
    xr = x.reshape(n, C)
    nblk = n // TOKEN_BLOCK

    S, e_row, k_row, psum, zsum = pl.pallas_call(
        _stage1_body,
        grid=(nblk,),
        in_specs=[
            pl.BlockSpec((TOKEN_BLOCK, C), lambda i: (i, 0)),
            pl.BlockSpec((E, C), lambda i: (0, 0)),
        ],
        out_specs=[
            pl.BlockSpec((TOKEN_BLOCK, E), lambda i: (i, 0)),
            pl.BlockSpec((1, 1, TOKEN_BLOCK), lambda i: (i, 0, 0)),
            pl.BlockSpec((1, 1, TOKEN_BLOCK), lambda i: (i, 0, 0)),
            pl.BlockSpec((1, E), lambda i: (0, 0)),
            pl.BlockSpec((1, 1), lambda i: (0, 0)),
        ],
        out_shape=[
            jax.ShapeDtypeStruct((n, E), jnp.int32),
            jax.ShapeDtypeStruct((nblk, 1, TOKEN_BLOCK), jnp.int32),
            jax.ShapeDtypeStruct((nblk, 1, TOKEN_BLOCK), jnp.int32),
            jax.ShapeDtypeStruct((1, E), jnp.float32),
            jax.ShapeDtypeStruct((1, 1), jnp.float32),
        ],
    )(xr, W)

    sc_out = pl.kernel(
        functools.partial(_stage2_sc_body, capacity, n),
        out_type=jax.ShapeDtypeStruct((32, 16), jnp.int32),
        mesh=plsc.VectorSubcoreMesh(core_axis_name="c", subcore_axis_name="s"),
        compiler_params=pltpu.CompilerParams(needs_layout_passes=False),
        scratch_types=[
            pltpu.VMEM((n,), jnp.int32),
            pltpu.VMEM((n,), jnp.int32),
            pltpu.VMEM((n + 16,), jnp.int32),
            pltpu.VMEM((n + 16,), jnp.int32),
            pltpu.VMEM((16,), jnp.int32),
        ],
    )(e_row.reshape(n), k_row.reshape(n))
    tprime = sc_out[:, 0:2].reshape(1, E)
    t = sc_out[:, 2:4].reshape(1, E)
    rem = sc_out[:, 4:6].reshape(1, E)
    kept = sc_out[:, 6:8].reshape(1, E)
    simple = sc_out[:, 8:10].reshape(1, E)

    mask, loss = pl.pallas_call(
        functools.partial(_stage3_body, n),
        grid=(nblk,),
        in_specs=[
            pl.BlockSpec((TOKEN_BLOCK, E), lambda i: (i, 0)),
            pl.BlockSpec((1, E), lambda i: (0, 0)),
            pl.BlockSpec((1, E), lambda i: (0, 0)),
            pl.BlockSpec((1, E), lambda i: (0, 0)),
            pl.BlockSpec((1, E), lambda i: (0, 0)),
            pl.BlockSpec((1, E), lambda i: (0, 0)),
            pl.BlockSpec((1, E), lambda i: (0, 0)),
            pl.BlockSpec((1, 1), lambda i: (0, 0)),
        ],
        out_specs=[
            pl.BlockSpec((TOKEN_BLOCK, E), lambda i: (i, 0)),
            pl.BlockSpec((1, 1), lambda i: (0, 0)),
        ],
        out_shape=[
            jax.ShapeDtypeStruct((n, E), jnp.float32),
            jax.ShapeDtypeStruct((1, 1), jnp.float32),
        ],
        scratch_shapes=[pltpu.VMEM((1, E), jnp.float32)],
    )(S, tprime, t, rem, kept, simple, psum, zsum)

    mask = mask.reshape(Bb, Tt, E)
    return mask, mask, loss.reshape(())


# speedup vs baseline: 1.4135x; 1.0419x over previous
import functools
import math

import jax
import jax.numpy as jnp
from jax import lax
from jax.experimental import pallas as pl
from jax.experimental.pallas import tpu as pltpu
from jax.experimental.pallas import tpu_sc as plsc

NUM_EXPERTS = 64
TOKEN_BLOCK = 1024
KEY_BIAS = 0x3C000000
AUX_W = 0.01
Z_W = 0.001


def _stage1_body(x_ref, w_ref, s_ref, e_ref, k_ref, psum_ref, zsum_ref):
    i = pl.program_id(0)
    logits = lax.dot_general(
        x_ref[...], w_ref[...], (((1,), (1,)), ((), ())),
        preferred_element_type=jnp.float32)
    m = jnp.max(logits, axis=1, keepdims=True)
    ex = jnp.exp(logits - m)
    ssum = jnp.sum(ex, axis=1, keepdims=True)
    p = ex / ssum
    ei = jnp.argmax(p, axis=1).astype(jnp.int32)
    pm = jnp.max(p, axis=1)
    bits = lax.bitcast_convert_type(pm, jnp.int32)
    key = jnp.maximum(bits - KEY_BIAS, 1)
    onehot = ei[:, None] == lax.broadcasted_iota(jnp.int32, (1, NUM_EXPERTS), 1)
    s_ref[...] = jnp.where(onehot, key[:, None], 0)
    e_ref[0, 0, :] = ei
    k_ref[0, 0, :] = key

    @pl.when(i == 0)
    def _():
        psum_ref[...] = jnp.zeros_like(psum_ref)
        zsum_ref[...] = jnp.zeros_like(zsum_ref)

    psum_ref[...] += jnp.sum(p, axis=0)[None, :]
    lse = m[:, 0] + jnp.log(ssum[:, 0])
    zsum_ref[...] += jnp.sum(lse * lse).reshape(1, 1)


def _sc_count_ge(list_ref, nchunks, u):
    def chunk(j, cnt):
        v = list_ref[pl.ds(j * 16, 16)]
        return cnt + plsc.all_reduce_population_count(v >= u)[0]
    return lax.fori_loop(0, nchunks, chunk, jnp.int32(0))


def _sc_search2(ref_a, n_a, ref_b, n_b, cap):
    nch_a = (n_a + 15) // 16
    nch_b = (n_b + 15) // 16
    def count(u):
        return _sc_count_ge(ref_a, nch_a, u) + _sc_count_ge(ref_b, nch_b, u)
    def bit(i, t):
        cand = t | (jnp.int32(1) << (25 - i))
        return jnp.where(count(cand) >= cap, cand, t)
    t = lax.fori_loop(0, 26, bit, jnp.int32(0))
    c_ge = count(t)
    c_gt = count(t + 1)
    rem = jnp.where(t > 0, cap - c_gt, 0)
    kept = jnp.minimum(n_a + n_b, cap)
    simple = jnp.where(t > 0, (c_ge == cap).astype(jnp.int32), 1)
    tprime = jnp.maximum(t, 1)
    return tprime, t, rem, kept, simple


def _stage2_sc_body(capacity, ntok, e_hbm, k_hbm, out_hbm,
                    e_v, k_v, l0a_v, l0b_v, l1a_v, l1b_v, outbuf_v):
    cap = jnp.int32(capacity)
    half = ntok // 2
    wid = lax.axis_index("c") * 16 + lax.axis_index("s")
    ex0 = (wid * 2).astype(jnp.int32)
    ex1 = ex0 + 1
    pltpu.sync_copy(e_hbm, e_v)
    pltpu.sync_copy(k_hbm, k_v)

    def compact(j, carry):
        ca0, ca1, cb0, cb1 = carry
        vea = e_v[pl.ds(j * 16, 16)]
        vka = k_v[pl.ds(j * 16, 16)]
        veb = e_v[pl.ds(half + j * 16, 16)]
        vkb = k_v[pl.ds(half + j * 16, 16)]
        ma0 = vea == ex0
        ma1 = vea == ex1
        mb0 = veb == ex0
        mb1 = veb == ex1
        plsc.store_compressed(l0a_v.at[pl.ds(ca0, 16)], vka, mask=ma0)
        plsc.store_compressed(l1a_v.at[pl.ds(ca1, 16)], vka, mask=ma1)
        plsc.store_compressed(l0b_v.at[pl.ds(cb0, 16)], vkb, mask=mb0)
        plsc.store_compressed(l1b_v.at[pl.ds(cb1, 16)], vkb, mask=mb1)
        ca0 = ca0 + plsc.all_reduce_population_count(ma0)[0]
        ca1 = ca1 + plsc.all_reduce_population_count(ma1)[0]
        cb0 = cb0 + plsc.all_reduce_population_count(mb0)[0]
        cb1 = cb1 + plsc.all_reduce_population_count(mb1)[0]
        return ca0, ca1, cb0, cb1

    z = jnp.int32(0)
    na0, na1, nb0, nb1 = lax.fori_loop(0, half // 16, compact, (z, z, z, z))
    zeros = jnp.zeros((16,), jnp.int32)
    l0a_v[pl.ds(na0, 16)] = zeros
    l1a_v[pl.ds(na1, 16)] = zeros
    l0b_v[pl.ds(nb0, 16)] = zeros
    l1b_v[pl.ds(nb1, 16)] = zeros
    tp0, t0, rem0, kept0, sim0 = _sc_search2(l0a_v, na0, l0b_v, nb0, cap)
    tp1, t1, rem1, kept1, sim1 = _sc_search2(l1a_v, na1, l1b_v, nb1, cap)

    lane = lax.iota(jnp.int32, 16)
    outv = jnp.where(lane == 0, tp0,
           jnp.where(lane == 1, tp1,
           jnp.where(lane == 2, t0,
           jnp.where(lane == 3, t1,
           jnp.where(lane == 4, rem0,
           jnp.where(lane == 5, rem1,
           jnp.where(lane == 6, kept0,
           jnp.where(lane == 7, kept1,
           jnp.where(lane == 8, sim0,
           jnp.where(lane == 9, sim1, 0))))))))))
    outbuf_v[...] = outv
    pltpu.sync_copy(outbuf_v, out_hbm.at[wid])


def _stage3_body(total_tokens, s_ref, tp_ref, t_ref, rem_ref,
                 kept_ref, sim_ref, psum_ref, zsum_ref,
                 mask_ref, loss_ref, carry_ref):
    i = pl.program_id(0)

    @pl.when(i == 0)
    def _():
        carry_ref[...] = jnp.zeros_like(carry_ref)

    S = s_ref[...]
    all_simple = jnp.min(sim_ref[...]) == 1

    @pl.when(all_simple)
    def _():
        mask_ref[...] = (S >= tp_ref[...]).astype(jnp.float32)

    @pl.when(jnp.logical_not(all_simple))
    def _():
        t = t_ref[...]
        gt = S > t
        eq = (S == t) & (t > 0)
        eqf = eq.astype(jnp.float32)
        row = lax.broadcasted_iota(jnp.int32, (TOKEN_BLOCK, TOKEN_BLOCK), 0)
        col = lax.broadcasted_iota(jnp.int32, (TOKEN_BLOCK, TOKEN_BLOCK), 1)
        tril = (row > col).astype(jnp.float32)
        excl = lax.dot_general(tril, eqf, (((1,), (0,)), ((), ())),
                               preferred_element_type=jnp.float32)
        rank = carry_ref[...] + excl
        keep_eq = eq & (rank < rem_ref[...].astype(jnp.float32))
        mask_ref[...] = (gt | keep_eq).astype(jnp.float32)
        carry_ref[...] += jnp.sum(eqf, axis=0)[None, :]

    @pl.when(i == pl.num_programs(0) - 1)
    def _():
        n = jnp.float32(total_tokens)
        f = kept_ref[...].astype(jnp.float32) / n
        pmean = psum_ref[...] / n
        aux = AUX_W * jnp.sum(f * pmean) * NUM_EXPERTS
        z = Z_W * zsum_ref[...] / n
        loss_ref[...] = aux + z


def kernel(x, W):
    Bb, Tt, C = x.shape
    E = W.shape[0]
    n = Bb * Tt
    capacity = math.ceil(n / E)
    xr = x.reshape(n, C)
    nblk = n // TOKEN_BLOCK

    S, e_row, k_row, psum, zsum = pl.pallas_call(
        _stage1_body,
        grid=(nblk,),
        compiler_params=pltpu.CompilerParams(vmem_limit_bytes=64 * 1024 * 1024),
        in_specs=[
            pl.BlockSpec((TOKEN_BLOCK, C), lambda i: (i, 0)),
            pl.BlockSpec((E, C), lambda i: (0, 0)),
        ],
        out_specs=[
            pl.BlockSpec((TOKEN_BLOCK, E), lambda i: (i, 0)),
            pl.BlockSpec((1, 1, TOKEN_BLOCK), lambda i: (i, 0, 0)),
            pl.BlockSpec((1, 1, TOKEN_BLOCK), lambda i: (i, 0, 0)),
            pl.BlockSpec((1, E), lambda i: (0, 0)),
            pl.BlockSpec((1, 1), lambda i: (0, 0)),
        ],
        out_shape=[
            jax.ShapeDtypeStruct((n, E), jnp.int32),
            jax.ShapeDtypeStruct((nblk, 1, TOKEN_BLOCK), jnp.int32),
            jax.ShapeDtypeStruct((nblk, 1, TOKEN_BLOCK), jnp.int32),
            jax.ShapeDtypeStruct((1, E), jnp.float32),
            jax.ShapeDtypeStruct((1, 1), jnp.float32),
        ],
    )(xr, W)

    sc_out = pl.kernel(
        functools.partial(_stage2_sc_body, capacity, n),
        out_type=jax.ShapeDtypeStruct((32, 16), jnp.int32),
        mesh=plsc.VectorSubcoreMesh(core_axis_name="c", subcore_axis_name="s"),
        compiler_params=pltpu.CompilerParams(needs_layout_passes=False),
        scratch_types=[
            pltpu.VMEM((n,), jnp.int32),
            pltpu.VMEM((n,), jnp.int32),
            pltpu.VMEM((n // 2 + 16,), jnp.int32),
            pltpu.VMEM((n // 2 + 16,), jnp.int32),
            pltpu.VMEM((n // 2 + 16,), jnp.int32),
            pltpu.VMEM((n // 2 + 16,), jnp.int32),
            pltpu.VMEM((16,), jnp.int32),
        ],
    )(e_row.reshape(n), k_row.reshape(n))
    tprime = sc_out[:, 0:2].reshape(1, E)
    t = sc_out[:, 2:4].reshape(1, E)
    rem = sc_out[:, 4:6].reshape(1, E)
    kept = sc_out[:, 6:8].reshape(1, E)
    simple = sc_out[:, 8:10].reshape(1, E)

    mask, loss = pl.pallas_call(
        functools.partial(_stage3_body, n),
        grid=(nblk,),
        in_specs=[
            pl.BlockSpec((TOKEN_BLOCK, E), lambda i: (i, 0)),
            pl.BlockSpec((1, E), lambda i: (0, 0)),
            pl.BlockSpec((1, E), lambda i: (0, 0)),
            pl.BlockSpec((1, E), lambda i: (0, 0)),
            pl.BlockSpec((1, E), lambda i: (0, 0)),
            pl.BlockSpec((1, E), lambda i: (0, 0)),
            pl.BlockSpec((1, E), lambda i: (0, 0)),
            pl.BlockSpec((1, 1), lambda i: (0, 0)),
        ],
        out_specs=[
            pl.BlockSpec((TOKEN_BLOCK, E), lambda i: (i, 0)),
            pl.BlockSpec((1, 1), lambda i: (0, 0)),
        ],
        out_shape=[
            jax.ShapeDtypeStruct((n, E), jnp.float32),
            jax.ShapeDtypeStruct((1, 1), jnp.float32),
        ],
        scratch_shapes=[pltpu.VMEM((1, E), jnp.float32)],
    )(S, tprime, t, rem, kept, simple, psum, zsum)

    mask = mask.reshape(Bb, Tt, E)
    return mask, mask, loss.reshape(())
